# baseline (speedup 1.0000x reference)
import functools
import math

import jax
import jax.numpy as jnp
from jax import lax
from jax.experimental import pallas as pl
from jax.experimental.pallas import tpu as pltpu
from jax.experimental.pallas import tpu_sc as plsc

NUM_BUCKETS = 32
MAX_DISTANCE = 128
NUM_HEADS = 16
QUERY_LEN = 2048
KEY_LEN = 2048

VW = 4096
W = 3968
P = 16
CLASSES = 8
UNITS_PER_TILE = 4
BLOCKS_PER_UNIT = 16
LANES = 16


def _build_v_kernel(tbl_ref, v_ref):
    half = NUM_BUCKETS // 2
    max_exact = half // 2
    scale = (half - max_exact) / math.log(MAX_DISTANCE / max_exact)
    h = pl.program_id(0)
    j_iota = lax.broadcasted_iota(jnp.int32, (NUM_BUCKETS, VW), 1)
    b_iota = lax.broadcasted_iota(jnp.int32, (NUM_BUCKETS, VW), 0)
    n = 2047 - j_iota
    ret = jnp.where(n < 0, half, 0)
    na = jnp.abs(n)
    is_small = na < max_exact
    safe = jnp.maximum(na, 1)
    log_val = (jnp.log(safe.astype(jnp.float32) / max_exact) * scale)
    log_val = log_val.astype(jnp.int32)
    bucket = jnp.where(is_small, na, max_exact + log_val)
    bucket = jnp.clip(bucket, 0, half - 1) + ret
    one_hot = (bucket == b_iota).astype(jnp.float32)
    rows = lax.dot_general(tbl_ref[...], one_hot, (((0,), (0,)), ((), ())),
                           precision=lax.Precision.HIGHEST,
                           preferred_element_type=jnp.float32)
    h_iota = lax.broadcasted_iota(jnp.int32, (NUM_HEADS, VW), 0)
    row = jnp.sum(jnp.where(h_iota == h, rows, 0.0), axis=0)
    v_ref[...] = row


def _build_v(bias_table):
    return pl.pallas_call(
        _build_v_kernel,
        grid=(NUM_HEADS,),
        in_specs=[pl.BlockSpec((NUM_BUCKETS, NUM_HEADS), lambda h: (0, 0))],
        out_specs=pl.BlockSpec((VW,), lambda h: (h,)),
        out_shape=jax.ShapeDtypeStruct((NUM_HEADS * VW,), jnp.float32),
    )(bias_table)


def _expand(v):
    mesh = plsc.VectorSubcoreMesh(core_axis_name="c", subcore_axis_name="s")

    @functools.partial(
        pl.kernel,
        out_type=jax.ShapeDtypeStruct((1, NUM_HEADS, QUERY_LEN, KEY_LEN),
                                      jnp.float32),
        mesh=mesh,
        scratch_types=[pltpu.VMEM((VW,), jnp.float32),
                       pltpu.VMEM((P, W), jnp.float32),
                       pltpu.VMEM((P, W), jnp.float32),
                       pltpu.SemaphoreType.DMA,
                       pltpu.SemaphoreType.DMA],
    )
    def expand_kernel(v_hbm, out_hbm, vbuf, buf0, buf1, sem0, sem1):
        wid = lax.axis_index("c") * 16 + lax.axis_index("s")
        h = wid // 2
        bufs = (buf0, buf1)
        sems = (sem0, sem1)
        pltpu.sync_copy(v_hbm.at[pl.ds(h * VW, VW)], vbuf)

        def drain_unit(b, sm):
            for _ in range(BLOCKS_PER_UNIT):
                pltpu.make_async_copy(b.at[:, pl.ds(0, KEY_LEN)],
                                      out_hbm.at[0, 0, pl.ds(0, P), :],
                                      sm).wait()

        for t in range(UNITS_PER_TILE):
            c = ((wid * UNITS_PER_TILE + t) % CLASSES) * P
            b = bufs[t % 2]
            sm = sems[t % 2]
            if t >= 2:
                drain_unit(b, sm)

            @pl.loop(0, W // LANES)
            def _(cc, b=b, c=c):
                col = cc * LANES
                for r in range(P):
                    start = col + (127 - c - r)
                    b[r, pl.ds(col, LANES)] = vbuf[pl.ds(start, LANES)]

            for m in range(BLOCKS_PER_UNIT):
                i0 = 1920 - 128 * m
                q0 = c + 128 * m
                pltpu.async_copy(b.at[:, pl.ds(i0, KEY_LEN)],
                                 out_hbm.at[0, h, pl.ds(q0, P), :], sm)

        for t in range(2):
            drain_unit(bufs[t], sems[t])

    return expand_kernel(v)


def kernel(query_len, key_len, bias_table):
    del query_len, key_len
    v = _build_v(bias_table)
    return _expand(v)


# speedup vs baseline: 1.0523x; 1.0523x over previous
import functools
import math

import jax
import jax.numpy as jnp
from jax import lax
from jax.experimental import pallas as pl
from jax.experimental.pallas import tpu as pltpu
from jax.experimental.pallas import tpu_sc as plsc

NUM_BUCKETS = 32
MAX_DISTANCE = 128
NUM_HEADS = 16
QUERY_LEN = 2048
KEY_LEN = 2048

VW = 4096
W = 3968
P = 16
CLASSES = 8
UNITS_PER_TILE = 4
BLOCKS_PER_UNIT = 16
LANES = 16


def _build_v_kernel(tbl_ref, v_ref):
    half = NUM_BUCKETS // 2
    max_exact = half // 2
    scale = (half - max_exact) / math.log(MAX_DISTANCE / max_exact)
    j_iota = lax.broadcasted_iota(jnp.int32, (NUM_BUCKETS, VW), 1)
    b_iota = lax.broadcasted_iota(jnp.int32, (NUM_BUCKETS, VW), 0)
    n = 2047 - j_iota
    ret = jnp.where(n < 0, half, 0)
    na = jnp.abs(n)
    is_small = na < max_exact
    safe = jnp.maximum(na, 1)
    log_val = (jnp.log(safe.astype(jnp.float32) / max_exact) * scale)
    log_val = log_val.astype(jnp.int32)
    bucket = jnp.where(is_small, na, max_exact + log_val)
    bucket = jnp.clip(bucket, 0, half - 1) + ret
    one_hot = (bucket == b_iota).astype(jnp.float32)
    v_ref[...] = lax.dot_general(tbl_ref[...], one_hot,
                                 (((0,), (0,)), ((), ())),
                                 precision=lax.Precision.HIGHEST,
                                 preferred_element_type=jnp.float32)


def _build_v(bias_table):
    v2 = pl.pallas_call(
        _build_v_kernel,
        out_shape=jax.ShapeDtypeStruct((NUM_HEADS, VW), jnp.float32),
    )(bias_table)
    return v2.reshape(NUM_HEADS * VW)


def _expand(v):
    mesh = plsc.VectorSubcoreMesh(core_axis_name="c", subcore_axis_name="s")

    @functools.partial(
        pl.kernel,
        out_type=jax.ShapeDtypeStruct((1, NUM_HEADS, QUERY_LEN, KEY_LEN),
                                      jnp.float32),
        mesh=mesh,
        scratch_types=[pltpu.VMEM((VW,), jnp.float32),
                       pltpu.VMEM((P, W), jnp.float32),
                       pltpu.VMEM((P, W), jnp.float32),
                       pltpu.SemaphoreType.DMA,
                       pltpu.SemaphoreType.DMA],
    )
    def expand_kernel(v_hbm, out_hbm, vbuf, buf0, buf1, sem0, sem1):
        wid = lax.axis_index("c") * 16 + lax.axis_index("s")
        h = wid // 2
        bufs = (buf0, buf1)
        sems = (sem0, sem1)
        pltpu.sync_copy(v_hbm.at[pl.ds(h * VW, VW)], vbuf)

        def drain_unit(b, sm):
            for _ in range(BLOCKS_PER_UNIT):
                pltpu.make_async_copy(b.at[:, pl.ds(0, KEY_LEN)],
                                      out_hbm.at[0, 0, pl.ds(0, P), :],
                                      sm).wait()

        for t in range(UNITS_PER_TILE):
            c = ((wid * UNITS_PER_TILE + t) % CLASSES) * P
            b = bufs[t % 2]
            sm = sems[t % 2]
            if t >= 2:
                drain_unit(b, sm)

            @pl.loop(0, W // LANES)
            def _(cc, b=b, c=c):
                col = cc * LANES
                for r in range(P):
                    start = col + (127 - c - r)
                    b[r, pl.ds(col, LANES)] = vbuf[pl.ds(start, LANES)]

            for m in range(BLOCKS_PER_UNIT):
                i0 = 1920 - 128 * m
                q0 = c + 128 * m
                pltpu.async_copy(b.at[:, pl.ds(i0, KEY_LEN)],
                                 out_hbm.at[0, h, pl.ds(q0, P), :], sm)

        for t in range(2):
            drain_unit(bufs[t], sems[t])

    return expand_kernel(v)


def kernel(query_len, key_len, bias_table):
    del query_len, key_len
    v = _build_v(bias_table)
    return _expand(v)


# speedup vs baseline: 1.0523x; 1.0000x over previous
import functools
import math

import jax
import jax.numpy as jnp
from jax import lax
from jax.experimental import pallas as pl
from jax.experimental.pallas import tpu as pltpu
from jax.experimental.pallas import tpu_sc as plsc

NUM_BUCKETS = 32
MAX_DISTANCE = 128
NUM_HEADS = 16
QUERY_LEN = 2048
KEY_LEN = 2048

VW = 4096
W = 3968
P = 16
CLASSES = 8
UNITS_PER_TILE = 4
BLOCKS_PER_UNIT = 16
LANES = 16


def _build_v_kernel(tbl_ref, v_ref):
    half = NUM_BUCKETS // 2
    max_exact = half // 2
    scale = (half - max_exact) / math.log(MAX_DISTANCE / max_exact)
    j_iota = lax.broadcasted_iota(jnp.int32, (NUM_BUCKETS, VW), 1)
    b_iota = lax.broadcasted_iota(jnp.int32, (NUM_BUCKETS, VW), 0)
    n = 2047 - j_iota
    ret = jnp.where(n < 0, half, 0)
    na = jnp.abs(n)
    is_small = na < max_exact
    safe = jnp.maximum(na, 1)
    log_val = (jnp.log(safe.astype(jnp.float32) / max_exact) * scale)
    log_val = log_val.astype(jnp.int32)
    bucket = jnp.where(is_small, na, max_exact + log_val)
    bucket = jnp.clip(bucket, 0, half - 1) + ret
    one_hot = (bucket == b_iota).astype(jnp.float32)
    v_ref[...] = lax.dot_general(tbl_ref[...], one_hot,
                                 (((0,), (0,)), ((), ())),
                                 precision=lax.Precision.HIGHEST,
                                 preferred_element_type=jnp.float32)


def _build_v(bias_table):
    v2 = pl.pallas_call(
        _build_v_kernel,
        out_shape=jax.ShapeDtypeStruct((NUM_HEADS, VW), jnp.float32),
    )(bias_table)
    return v2.reshape(NUM_HEADS * VW)


def _expand(v):
    mesh = plsc.VectorSubcoreMesh(core_axis_name="c", subcore_axis_name="s")

    @functools.partial(
        pl.kernel,
        out_type=jax.ShapeDtypeStruct((1, NUM_HEADS, QUERY_LEN, KEY_LEN),
                                      jnp.float32),
        mesh=mesh,
        scratch_types=[pltpu.VMEM((VW,), jnp.float32),
                       pltpu.VMEM((P, W), jnp.float32),
                       pltpu.VMEM((P, W), jnp.float32),
                       pltpu.SemaphoreType.DMA,
                       pltpu.SemaphoreType.DMA],
    )
    def expand_kernel(v_hbm, out_hbm, vbuf, buf0, buf1, sem0, sem1):
        wid = lax.axis_index("c") * 16 + lax.axis_index("s")
        h = wid // 2
        bufs = (buf0, buf1)
        sems = (sem0, sem1)
        pltpu.sync_copy(v_hbm.at[pl.ds(h * VW, VW)], vbuf)

        def drain_unit(b, sm):
            for _ in range(BLOCKS_PER_UNIT):
                pltpu.make_async_copy(b.at[:, pl.ds(0, KEY_LEN)],
                                      out_hbm.at[0, 0, pl.ds(0, P), :],
                                      sm).wait()

        for t in range(UNITS_PER_TILE):
            c = ((wid * UNITS_PER_TILE + t) % CLASSES) * P
            b = bufs[t % 2]
            sm = sems[t % 2]
            if t >= 2:
                drain_unit(b, sm)

            @pl.loop(0, W // (2 * LANES))
            def _(cc, b=b, c=c):
                for j in range(2):
                    col = cc * (2 * LANES) + j * LANES
                    for r in range(P):
                        start = col + (127 - c - r)
                        b[r, pl.ds(col, LANES)] = vbuf[pl.ds(start, LANES)]

            for m in range(BLOCKS_PER_UNIT):
                i0 = 1920 - 128 * m
                q0 = c + 128 * m
                pltpu.async_copy(b.at[:, pl.ds(i0, KEY_LEN)],
                                 out_hbm.at[0, h, pl.ds(q0, P), :], sm)

        for t in range(2):
            drain_unit(bufs[t], sems[t])

    return expand_kernel(v)


def kernel(query_len, key_len, bias_table):
    del query_len, key_len
    v = _build_v(bias_table)
    return _expand(v)


# speedup vs baseline: 1.0592x; 1.0066x over previous
import functools
import math

import jax
import jax.numpy as jnp
from jax import lax
from jax.experimental import pallas as pl
from jax.experimental.pallas import tpu as pltpu
from jax.experimental.pallas import tpu_sc as plsc

NUM_BUCKETS = 32
MAX_DISTANCE = 128
NUM_HEADS = 16
QUERY_LEN = 2048
KEY_LEN = 2048

VW = 4096
W = 3968
P = 16
CLASSES = 8
UNITS_PER_TILE = 4
BLOCKS_PER_UNIT = 16
LANES = 16


def _build_v_kernel(tbl_ref, v_ref):
    half = NUM_BUCKETS // 2
    max_exact = half // 2
    scale = (half - max_exact) / math.log(MAX_DISTANCE / max_exact)
    j_iota = lax.broadcasted_iota(jnp.int32, (NUM_BUCKETS, VW), 1)
    b_iota = lax.broadcasted_iota(jnp.int32, (NUM_BUCKETS, VW), 0)
    n = 2047 - j_iota
    ret = jnp.where(n < 0, half, 0)
    na = jnp.abs(n)
    is_small = na < max_exact
    safe = jnp.maximum(na, 1)
    log_val = (jnp.log(safe.astype(jnp.float32) / max_exact) * scale)
    log_val = log_val.astype(jnp.int32)
    bucket = jnp.where(is_small, na, max_exact + log_val)
    bucket = jnp.clip(bucket, 0, half - 1) + ret
    one_hot = (bucket == b_iota).astype(jnp.float32)
    v_ref[...] = lax.dot_general(tbl_ref[...], one_hot,
                                 (((0,), (0,)), ((), ())),
                                 precision=lax.Precision.HIGHEST,
                                 preferred_element_type=jnp.float32)


def _build_v(bias_table):
    v2 = pl.pallas_call(
        _build_v_kernel,
        out_shape=jax.ShapeDtypeStruct((NUM_HEADS, VW), jnp.float32),
    )(bias_table)
    return v2.reshape(NUM_HEADS * VW)


def _expand(v):
    mesh = plsc.VectorSubcoreMesh(core_axis_name="c", subcore_axis_name="s")

    @functools.partial(
        pl.kernel,
        out_type=jax.ShapeDtypeStruct((1, NUM_HEADS, QUERY_LEN, KEY_LEN),
                                      jnp.float32),
        mesh=mesh,
        scratch_types=[pltpu.VMEM((VW,), jnp.float32),
                       pltpu.VMEM((P, W), jnp.float32),
                       pltpu.VMEM((P, W), jnp.float32),
                       pltpu.SemaphoreType.DMA,
                       pltpu.SemaphoreType.DMA],
    )
    def expand_kernel(v_hbm, out_hbm, vbuf, buf0, buf1, sem0, sem1):
        wid = lax.axis_index("c") * 16 + lax.axis_index("s")
        h = wid // 2
        bufs = (buf0, buf1)
        sems = (sem0, sem1)
        pltpu.sync_copy(v_hbm.at[pl.ds(h * VW, VW)], vbuf)

        def drain_unit(b, sm):
            for _ in range(BLOCKS_PER_UNIT):
                pltpu.make_async_copy(b.at[:, pl.ds(0, KEY_LEN)],
                                      out_hbm.at[0, 0, pl.ds(0, P), :],
                                      sm).wait()

        for t in range(UNITS_PER_TILE):
            c = ((wid * UNITS_PER_TILE + t) % CLASSES) * P
            b = bufs[t % 2]
            sm = sems[t % 2]
            if t >= 2:
                drain_unit(b, sm)

            @pl.loop(0, W // LANES)
            def _(cc, b=b, c=c):
                col = cc * LANES
                for r in range(P):
                    start = col + (127 - c - r)
                    b[r, pl.ds(col, LANES)] = vbuf[pl.ds(start, LANES)]

            for m in range(BLOCKS_PER_UNIT):
                i0 = 1920 - 128 * m
                q0 = c + 128 * m
                pltpu.async_copy(b.at[:, pl.ds(i0, KEY_LEN)],
                                 out_hbm.at[0, h, pl.ds(q0, P), :], sm)

        for t in range(2):
            drain_unit(bufs[t], sems[t])

    return expand_kernel(v)


def kernel(query_len, key_len, bias_table):
    del query_len, key_len
    v = _build_v(bias_table)
    return _expand(v)
